# baseline (speedup 1.0000x reference)
import functools

import jax
import jax.numpy as jnp
from jax import lax
from jax.experimental import pallas as pl
from jax.experimental.pallas import tpu as pltpu
from jax.experimental.pallas import tpu_sc as plsc


def _make_lookup(V, D, B):
    info = plsc.get_sparse_core_info()
    NC, NS = info.num_cores, info.num_subcores
    NW = NC * NS
    b_per_w = B // NW
    CH = 128
    n_chunks = b_per_w // CH

    mesh = plsc.VectorSubcoreMesh(core_axis_name="c", subcore_axis_name="s")

    @functools.partial(
        pl.kernel,
        mesh=mesh,
        out_type=jax.ShapeDtypeStruct((B, D), jnp.float32),
        compiler_params=pltpu.CompilerParams(use_tc_tiling_on_sc=False),
        scratch_types=[
            pltpu.VMEM((n_chunks, CH), jnp.int32),
            pltpu.VMEM((b_per_w, D), jnp.float32),
            pltpu.SemaphoreType.DMA,
        ],
    )
    def lookup(idx_hbm, table_hbm, out_hbm, idx_v, rows_v, sem):
        wid = lax.axis_index("s") * NC + lax.axis_index("c")
        base = wid * b_per_w
        pltpu.sync_copy(idx_hbm.at[pl.ds(wid * n_chunks, n_chunks)], idx_v)
        copies = []
        for j in range(n_chunks):
            copies.append(
                pltpu.async_copy(
                    table_hbm.at[idx_v.at[j]],
                    rows_v.at[pl.ds(j * CH, CH)],
                    sem,
                )
            )
        for c in copies:
            c.wait()
        pltpu.sync_copy(rows_v, out_hbm.at[pl.ds(base, b_per_w)])

    return lookup


def kernel(x, weight):
    B = x.shape[0]
    V, D = weight.shape
    idx = x.astype(jnp.int32).reshape(B // 128, 128)
    return _make_lookup(V, D, B)(idx, weight)


# speedup vs baseline: 4.2729x; 4.2729x over previous
import functools

import jax
import jax.numpy as jnp
from jax import lax
from jax.experimental import pallas as pl
from jax.experimental.pallas import tpu as pltpu
from jax.experimental.pallas import tpu_sc as plsc


def _make_lookup(V, D, B):
    info = plsc.get_sparse_core_info()
    NC, NS, L = info.num_cores, info.num_subcores, info.num_lanes
    NW = NC * NS
    b_per_w = B // NW
    G = 128
    n_groups = b_per_w // G
    SB = 8
    n_sb = G // SB
    rows_per_w = b_per_w // L

    TAIL0 = (V // 128) * 128
    tail_len = V - TAIL0
    LAST_AL = TAIL0 - 128

    mesh = plsc.VectorSubcoreMesh(core_axis_name="c", subcore_axis_name="s")

    @functools.partial(
        pl.kernel,
        mesh=mesh,
        out_type=jax.ShapeDtypeStruct((D, B), jnp.float32),
        compiler_params=pltpu.CompilerParams(needs_layout_passes=False),
        scratch_types=[
            pltpu.VMEM((rows_per_w, L), jnp.int32),
            pltpu.VMEM((tail_len, D), jnp.float32),
            pltpu.VMEM((2, SB, D, 128), jnp.float32),
            pltpu.VMEM((D, G), jnp.float32),
            pltpu.SemaphoreType.DMA,
            pltpu.SemaphoreType.DMA,
        ],
    )
    def lookup(idx_hbm, wt_hbm, tail_hbm, out_hbm, idx2, tail_v, slots, bufg,
               sem, wsem):
        wid = lax.axis_index("s") * NC + lax.axis_index("c")
        base = wid * b_per_w
        lanes16 = lax.iota(jnp.int32, L)
        pltpu.sync_copy(idx_hbm.at[pl.ds(wid * rows_per_w, rows_per_w), :], idx2)
        pltpu.sync_copy(tail_hbm, tail_v)

        def read_scalars(g, sb):
            vec = idx2[g * (G // L) + sb // 2, :]
            out = []
            for jj in range(SB):
                lane = (sb % 2) * SB + jj
                out.append(
                    lax.reduce_max(
                        jnp.where(lanes16 == lane, vec, 0), axes=(0,)
                    )
                )
            return out

        def fire(bank, rs):
            for jj, r in enumerate(rs):
                r_al = pl.multiple_of(
                    jnp.minimum((r // 128) * 128, LAST_AL), 128
                )
                pltpu.async_copy(
                    wt_hbm.at[:, pl.ds(r_al, 128)],
                    slots.at[bank, jj],
                    sem,
                )

        def extract(bank, sb, rs):
            for jj, r in enumerate(rs):
                pltpu.make_async_copy(
                    wt_hbm.at[:, pl.ds(0, 128)], slots.at[bank, jj], sem
                ).wait()
                lane = jnp.full((L,), r % 128, jnp.int32)
                tail_row = jnp.full(
                    (L,), jnp.clip(r - TAIL0, 0, tail_len - 1), jnp.int32
                )
                is_tail = jnp.full((L,), r >= TAIL0, jnp.bool_)
                col = jnp.full((L,), sb * SB + jj, jnp.int32)
                for h in range(D // L):
                    feats = lanes16 + h * L
                    vals = plsc.load_gather(slots.at[bank, jj], [feats, lane])
                    tvals = plsc.load_gather(tail_v, [tail_row, feats])
                    vals = jnp.where(is_tail, tvals, vals)
                    plsc.store_scatter(bufg, [feats, col], vals)

        def group(g, _):
            rs = read_scalars(g, 0)
            fire(0, rs)
            for sb in range(n_sb):
                if sb + 1 < n_sb:
                    rs_next = read_scalars(g, sb + 1)
                    fire((sb + 1) % 2, rs_next)
                else:
                    rs_next = None
                extract(sb % 2, sb, rs)
                rs = rs_next
            gcol = pl.multiple_of(base + g * G, 128)
            cp = pltpu.make_async_copy(bufg, out_hbm.at[:, pl.ds(gcol, G)], wsem)
            cp.start()
            cp.wait()
            return 0

        lax.fori_loop(0, n_groups, group, 0)

    return lookup


def kernel(x, weight):
    B = x.shape[0]
    V, D = weight.shape
    L = 16
    idx2 = x.astype(jnp.int32).reshape(B // L, L)
    tail = weight[(V // 128) * 128:]
    out_t = _make_lookup(V, D, B)(idx2, weight.T, tail)
    return out_t.T
